# baseline (speedup 1.0000x reference)
import functools

import jax
import jax.numpy as jnp
from jax import lax
from jax.experimental import pallas as pl
from jax.experimental.pallas import tpu as pltpu
from jax.experimental.pallas import tpu_sc as plsc

NC = 2
NS = 16
NW = NC * NS
CH = 128


def _deg_kernel(n_pad, cpt):
  zslice = n_pad // NS
  mesh = plsc.VectorSubcoreMesh(core_axis_name="c", subcore_axis_name="s")

  @functools.partial(
      pl.kernel,
      out_type=jax.ShapeDtypeStruct((NC * 2 * n_pad,), jnp.float32),
      mesh=mesh,
      scratch_types=[
          pltpu.VMEM((zslice,), jnp.float32),
          pltpu.VMEM((CH,), jnp.float32),
          pltpu.VMEM((cpt * CH,), jnp.int32),
          pltpu.VMEM((cpt * CH,), jnp.int32),
          pltpu.VMEM_SHARED((n_pad,), jnp.float32),
          pltpu.VMEM_SHARED((n_pad,), jnp.float32),
      ],
  )
  def k(src_hbm, dst_hbm, ones_hbm, z_hbm, out_hbm, zbuf, ones_v, sidx,
        didx, hist_o, hist_i):
    c = lax.axis_index("c")
    s = lax.axis_index("s")
    wid = s * NC + c
    ebase = wid * cpt * CH

    pltpu.sync_copy(z_hbm, zbuf)
    pltpu.sync_copy(zbuf, hist_o.at[pl.ds(s * zslice, zslice)])
    pltpu.sync_copy(zbuf, hist_i.at[pl.ds(s * zslice, zslice)])
    pltpu.sync_copy(ones_hbm, ones_v)
    pltpu.sync_copy(src_hbm.at[pl.ds(ebase, cpt * CH)], sidx)
    pltpu.sync_copy(dst_hbm.at[pl.ds(ebase, cpt * CH)], didx)
    plsc.subcore_barrier()

    @pl.loop(0, cpt)
    def _(j):
      pltpu.sync_copy(ones_v, hist_o.at[sidx.at[pl.ds(j * CH, CH)]],
                      add=True)
      pltpu.sync_copy(ones_v, hist_i.at[didx.at[pl.ds(j * CH, CH)]],
                      add=True)

    plsc.subcore_barrier()

    pltpu.sync_copy(hist_o.at[pl.ds(s * zslice, zslice)], zbuf)
    pltpu.sync_copy(zbuf, out_hbm.at[pl.ds((c * 2) * n_pad + s * zslice,
                                           zslice)])
    pltpu.sync_copy(hist_i.at[pl.ds(s * zslice, zslice)], zbuf)
    pltpu.sync_copy(zbuf, out_hbm.at[pl.ds((c * 2 + 1) * n_pad + s * zslice,
                                           zslice)])

  return k


def _agg_kernel(n_pad, d, cpt0, cpt1):
  zrows = n_pad // NS
  mesh = plsc.VectorSubcoreMesh(core_axis_name="c", subcore_axis_name="s")

  @functools.partial(
      pl.kernel,
      out_type=jax.ShapeDtypeStruct((NC * n_pad, d), jnp.float32),
      mesh=mesh,
      scratch_types=[
          [pltpu.VMEM((CH,), jnp.int32) for _ in range(4)],
          [pltpu.VMEM((CH,), jnp.int32) for _ in range(4)],
          [pltpu.VMEM((CH, d), jnp.float32) for _ in range(2)],
          pltpu.VMEM_SHARED((n_pad, d), jnp.float32),
          pltpu.SemaphoreType.DMA,
          [pltpu.SemaphoreType.DMA for _ in range(2)],
      ],
  )
  def k(h_hbm, src_hbm, dst_hbm, z_hbm, out_hbm, sidx, didx, rows, acc,
        lsem, gsem):
    c = lax.axis_index("c")
    s = lax.axis_index("s")
    cptc = jnp.where(c == 0, cpt0, cpt1)
    cbase = jnp.where(c == 0, s * cpt0, NS * cpt0 + s * cpt1)

    pltpu.sync_copy(z_hbm, rows[0])
    for r in range(zrows // CH):
      pltpu.sync_copy(rows[0], acc.at[pl.ds(s * zrows + r * CH, CH)])

    def load_idx(j, a, b):
      lds = []
      for i, q in enumerate(range(a, b + 1)):
        off = (cbase + j + i) * CH
        lds.append(pltpu.async_copy(src_hbm.at[pl.ds(off, CH)], sidx[q],
                                    lsem))
        lds.append(pltpu.async_copy(dst_hbm.at[pl.ds(off, CH)], didx[q],
                                    lsem))
      return lds

    for d_ in load_idx(0, 0, 1):
      d_.wait()
    plsc.subcore_barrier()

    @pl.loop(0, cptc // 4)
    def _(g):
      j = g * 4
      for ph in range(2):
        q0, q1 = (0, 1) if ph == 0 else (2, 3)
        n0, n1 = (2, 3) if ph == 0 else (0, 1)
        g0 = pltpu.async_copy(h_hbm.at[sidx[q0]], rows[0], gsem[0])
        g1 = pltpu.async_copy(h_hbm.at[sidx[q1]], rows[1], gsem[1])
        lds = load_idx(j + 2 * ph + 2, n0, n1)
        for d_ in lds:
          d_.wait()
        g0.wait()
        g1.wait()
        pltpu.sync_copy(rows[0], acc.at[didx[q0]], add=True)
        pltpu.sync_copy(rows[1], acc.at[didx[q1]], add=True)

    plsc.subcore_barrier()

    for r in range(zrows // CH):
      roff = s * zrows + r * CH
      pltpu.sync_copy(acc.at[pl.ds(roff, CH)], rows[0])
      pltpu.sync_copy(rows[0], out_hbm.at[pl.ds(c * n_pad + roff, CH)])

  return k


def _mm_body(x_ref, w_ref, y_ref):
  y_ref[...] = jnp.dot(x_ref[...], w_ref[...],
                       preferred_element_type=jnp.float32)


def _norms_body(degs_ref, out_ref):
  dego = degs_ref[0, 0] + degs_ref[1, 0]
  degi = degs_ref[0, 1] + degs_ref[1, 1]
  out_ref[0] = lax.rsqrt(jnp.maximum(dego, 1.0))
  out_ref[1] = lax.rsqrt(jnp.maximum(degi, 1.0))


def _scale_body(y_ref, ns_ref, h_ref):
  h_ref[...] = y_ref[...] * ns_ref[...]


def _final_body(acc_ref, nd_ref, b_ref, out_ref, *, n, n_pad):
  agg = acc_ref[0:n] + acc_ref[n_pad:n_pad + n]
  y = agg * nd_ref[...] + b_ref[...]
  m = jnp.max(y, axis=1, keepdims=True)
  e = jnp.exp(y - m)
  out_ref[...] = e / jnp.sum(e, axis=1, keepdims=True)


def kernel(x, edge_index, W, b):
  n, d = x.shape
  e = edge_index.shape[1]
  n_pad = ((n + 1 + NS * CH - 1) // (NS * CH)) * (NS * CH)
  cpt = 2 * -(-e // (NW * CH * 2))
  e_pad = NW * CH * cpt

  cpt0, cpt1 = 56, 2 * cpt - 56
  assert cpt0 % 4 == 0 and cpt1 % 4 == 0 and cpt0 + cpt1 == 2 * cpt

  src = edge_index[0].astype(jnp.int32)
  dst = edge_index[1].astype(jnp.int32)
  pad = e_pad - e + 2 * CH
  if pad:
    padv = n + jnp.arange(pad, dtype=jnp.int32) % (n_pad - n)
    src = jnp.concatenate([src, padv])
    dst = jnp.concatenate([dst, padv])

  ones_e = jnp.ones((CH,), jnp.float32)
  z1 = jnp.zeros((n_pad // NS,), jnp.float32)
  z2 = jnp.zeros((CH, d), jnp.float32)

  degs = _deg_kernel(n_pad, cpt)(src, dst, ones_e, z1)
  x_pad = jnp.concatenate([x, jnp.zeros((n_pad - n, d), jnp.float32)])
  y = pl.pallas_call(
      _mm_body, out_shape=jax.ShapeDtypeStruct((n_pad, d), jnp.float32))(
          x_pad, W)

  degs4 = degs.reshape(NC, 2, n_pad // d, d)
  norms = pl.pallas_call(
      _norms_body,
      out_shape=jax.ShapeDtypeStruct((2, n_pad // d, d), jnp.float32))(degs4)
  ns_col = norms[0].reshape(-1)[:, None]
  nd_col = norms[1].reshape(-1)[:n][:, None]

  h = pl.pallas_call(
      _scale_body,
      out_shape=jax.ShapeDtypeStruct((n_pad, d), jnp.float32))(y, ns_col)

  acc = _agg_kernel(n_pad, d, cpt0, cpt1)(h, src, dst, z2)

  out = pl.pallas_call(
      functools.partial(_final_body, n=n, n_pad=n_pad),
      out_shape=jax.ShapeDtypeStruct((n, d), jnp.float32))(
          acc, nd_col, b.reshape(1, d))
  return out


# speedup vs baseline: 1.1591x; 1.1591x over previous
import functools

import jax
import jax.numpy as jnp
from jax import lax
from jax.experimental import pallas as pl
from jax.experimental.pallas import tpu as pltpu
from jax.experimental.pallas import tpu_sc as plsc

NC = 2
NS = 16
NW = NC * NS
CH = 128


def _deg_kernel(n_pad, cpt):
  zslice = n_pad // NS
  mesh = plsc.VectorSubcoreMesh(core_axis_name="c", subcore_axis_name="s")

  @functools.partial(
      pl.kernel,
      out_type=jax.ShapeDtypeStruct((NC * 2 * n_pad,), jnp.float32),
      mesh=mesh,
      scratch_types=[
          pltpu.VMEM((zslice,), jnp.float32),
          pltpu.VMEM((CH,), jnp.float32),
          pltpu.VMEM((cpt * CH,), jnp.int32),
          pltpu.VMEM((cpt * CH,), jnp.int32),
          pltpu.VMEM_SHARED((n_pad,), jnp.float32),
          pltpu.VMEM_SHARED((n_pad,), jnp.float32),
      ],
  )
  def k(src_hbm, dst_hbm, ones_hbm, z_hbm, out_hbm, zbuf, ones_v, sidx,
        didx, hist_o, hist_i):
    c = lax.axis_index("c")
    s = lax.axis_index("s")
    wid = s * NC + c
    ebase = wid * cpt * CH

    pltpu.sync_copy(z_hbm, zbuf)
    pltpu.sync_copy(zbuf, hist_o.at[pl.ds(s * zslice, zslice)])
    pltpu.sync_copy(zbuf, hist_i.at[pl.ds(s * zslice, zslice)])
    pltpu.sync_copy(ones_hbm, ones_v)
    pltpu.sync_copy(src_hbm.at[pl.ds(ebase, cpt * CH)], sidx)
    pltpu.sync_copy(dst_hbm.at[pl.ds(ebase, cpt * CH)], didx)
    plsc.subcore_barrier()

    @pl.loop(0, cpt)
    def _(j):
      pltpu.sync_copy(ones_v, hist_o.at[sidx.at[pl.ds(j * CH, CH)]],
                      add=True)
      pltpu.sync_copy(ones_v, hist_i.at[didx.at[pl.ds(j * CH, CH)]],
                      add=True)

    plsc.subcore_barrier()

    pltpu.sync_copy(hist_o.at[pl.ds(s * zslice, zslice)], zbuf)
    pltpu.sync_copy(zbuf, out_hbm.at[pl.ds((c * 2) * n_pad + s * zslice,
                                           zslice)])
    pltpu.sync_copy(hist_i.at[pl.ds(s * zslice, zslice)], zbuf)
    pltpu.sync_copy(zbuf, out_hbm.at[pl.ds((c * 2 + 1) * n_pad + s * zslice,
                                           zslice)])

  return k


def _agg_kernel(n_pad, d, cpt0, cpt1):
  zrows = n_pad // NS
  mesh = plsc.VectorSubcoreMesh(core_axis_name="c", subcore_axis_name="s")

  @functools.partial(
      pl.kernel,
      out_type=jax.ShapeDtypeStruct((NC * n_pad, d), jnp.float32),
      mesh=mesh,
      scratch_types=[
          [pltpu.VMEM((CH,), jnp.int32) for _ in range(4)],
          [pltpu.VMEM((CH,), jnp.int32) for _ in range(4)],
          [pltpu.VMEM((CH, d), jnp.float32) for _ in range(2)],
          pltpu.VMEM_SHARED((n_pad, d), jnp.float32),
          pltpu.SemaphoreType.DMA,
          [pltpu.SemaphoreType.DMA for _ in range(2)],
      ],
  )
  def k(h_hbm, src_hbm, dst_hbm, z_hbm, out_hbm, sidx, didx, rows, acc,
        lsem, gsem):
    c = lax.axis_index("c")
    s = lax.axis_index("s")
    cptc = jnp.where(c == 0, cpt0, cpt1)
    cbase = jnp.where(c == 0, s * cpt0, NS * cpt0 + s * cpt1)

    pltpu.sync_copy(z_hbm, rows[0])
    for r in range(zrows // CH):
      pltpu.sync_copy(rows[0], acc.at[pl.ds(s * zrows + r * CH, CH)])

    def load_idx(j, a, b):
      lds = []
      for i, q in enumerate(range(a, b + 1)):
        off = (cbase + j + i) * CH
        lds.append(pltpu.async_copy(src_hbm.at[pl.ds(off, CH)], sidx[q],
                                    lsem))
        lds.append(pltpu.async_copy(dst_hbm.at[pl.ds(off, CH)], didx[q],
                                    lsem))
      return lds

    for d_ in load_idx(0, 0, 1):
      d_.wait()
    plsc.subcore_barrier()

    @pl.loop(0, cptc // 4)
    def _(g):
      j = g * 4
      for ph in range(2):
        q0, q1 = (0, 1) if ph == 0 else (2, 3)
        n0, n1 = (2, 3) if ph == 0 else (0, 1)
        g0 = pltpu.async_copy(h_hbm.at[sidx[q0]], rows[0], gsem[0])
        g1 = pltpu.async_copy(h_hbm.at[sidx[q1]], rows[1], gsem[1])
        lds = load_idx(j + 2 * ph + 2, n0, n1)
        for d_ in lds:
          d_.wait()
        g0.wait()
        g1.wait()
        pltpu.sync_copy(rows[0], acc.at[didx[q0]], add=True)
        pltpu.sync_copy(rows[1], acc.at[didx[q1]], add=True)

    plsc.subcore_barrier()

    for r in range(zrows // CH):
      roff = s * zrows + r * CH
      pltpu.sync_copy(acc.at[pl.ds(roff, CH)], rows[0])
      pltpu.sync_copy(rows[0], out_hbm.at[pl.ds(c * n_pad + roff, CH)])

  return k


def _mm_body(x_ref, w_ref, y_ref):
  y_ref[...] = jnp.dot(x_ref[...], w_ref[...],
                       preferred_element_type=jnp.float32)


def _norms_body(degs_ref, out_ref):
  dego = degs_ref[0, 0] + degs_ref[1, 0]
  degi = degs_ref[0, 1] + degs_ref[1, 1]
  out_ref[0] = lax.rsqrt(jnp.maximum(dego, 1.0))
  out_ref[1] = lax.rsqrt(jnp.maximum(degi, 1.0))


def _scale_body(y_ref, ns_ref, h_ref):
  h_ref[...] = y_ref[...] * ns_ref[...]


def _final_body(acc_ref, nd_ref, b_ref, out_ref, *, n, n_pad):
  agg = acc_ref[0:n] + acc_ref[n_pad:n_pad + n]
  y = agg * nd_ref[...] + b_ref[...]
  m = jnp.max(y, axis=1, keepdims=True)
  e = jnp.exp(y - m)
  out_ref[...] = e / jnp.sum(e, axis=1, keepdims=True)


def kernel(x, edge_index, W, b):
  n, d = x.shape
  e = edge_index.shape[1]
  n_pad = ((n + 1 + NS * CH - 1) // (NS * CH)) * (NS * CH)
  cpt = 2 * -(-e // (NW * CH * 2))
  e_pad = NW * CH * cpt

  cpt0, cpt1 = cpt, cpt
  assert cpt0 % 4 == 0 and cpt1 % 4 == 0 and cpt0 + cpt1 == 2 * cpt

  src = edge_index[0].astype(jnp.int32)
  dst = edge_index[1].astype(jnp.int32)
  pad = e_pad - e + 2 * CH
  if pad:
    padv = n + jnp.arange(pad, dtype=jnp.int32) % (n_pad - n)
    src = jnp.concatenate([src, padv])
    dst = jnp.concatenate([dst, padv])

  ones_e = jnp.ones((CH,), jnp.float32)
  z1 = jnp.zeros((n_pad // NS,), jnp.float32)
  z2 = jnp.zeros((CH, d), jnp.float32)

  degs = _deg_kernel(n_pad, cpt)(src, dst, ones_e, z1)
  x_pad = jnp.concatenate([x, jnp.zeros((n_pad - n, d), jnp.float32)])
  y = pl.pallas_call(
      _mm_body, out_shape=jax.ShapeDtypeStruct((n_pad, d), jnp.float32))(
          x_pad, W)

  degs4 = degs.reshape(NC, 2, n_pad // d, d)
  norms = pl.pallas_call(
      _norms_body,
      out_shape=jax.ShapeDtypeStruct((2, n_pad // d, d), jnp.float32))(degs4)
  ns_col = norms[0].reshape(-1)[:, None]
  nd_col = norms[1].reshape(-1)[:n][:, None]

  h = pl.pallas_call(
      _scale_body,
      out_shape=jax.ShapeDtypeStruct((n_pad, d), jnp.float32))(y, ns_col)

  acc = _agg_kernel(n_pad, d, cpt0, cpt1)(h, src, dst, z2)

  out = pl.pallas_call(
      functools.partial(_final_body, n=n, n_pad=n_pad),
      out_shape=jax.ShapeDtypeStruct((n, d), jnp.float32))(
          acc, nd_col, b.reshape(1, d))
  return out


# speedup vs baseline: 1.1724x; 1.0114x over previous
import functools

import jax
import jax.numpy as jnp
from jax import lax
from jax.experimental import pallas as pl
from jax.experimental.pallas import tpu as pltpu
from jax.experimental.pallas import tpu_sc as plsc

NC = 2
NS = 16
NW = NC * NS
CH = 128


def _deg_kernel(n_pad, cpt):
  zslice = n_pad // NS
  mesh = plsc.VectorSubcoreMesh(core_axis_name="c", subcore_axis_name="s")

  @functools.partial(
      pl.kernel,
      out_type=jax.ShapeDtypeStruct((NC * 2 * n_pad,), jnp.float32),
      mesh=mesh,
      scratch_types=[
          pltpu.VMEM((zslice,), jnp.float32),
          pltpu.VMEM((CH,), jnp.float32),
          pltpu.VMEM((cpt * CH,), jnp.int32),
          pltpu.VMEM((cpt * CH,), jnp.int32),
          pltpu.VMEM_SHARED((n_pad,), jnp.float32),
          pltpu.VMEM_SHARED((n_pad,), jnp.float32),
      ],
  )
  def k(src_hbm, dst_hbm, ones_hbm, z_hbm, out_hbm, zbuf, ones_v, sidx,
        didx, hist_o, hist_i):
    c = lax.axis_index("c")
    s = lax.axis_index("s")
    wid = s * NC + c
    ebase = wid * cpt * CH

    pltpu.sync_copy(z_hbm, zbuf)
    pltpu.sync_copy(zbuf, hist_o.at[pl.ds(s * zslice, zslice)])
    pltpu.sync_copy(zbuf, hist_i.at[pl.ds(s * zslice, zslice)])
    pltpu.sync_copy(ones_hbm, ones_v)
    pltpu.sync_copy(src_hbm.at[pl.ds(ebase, cpt * CH)], sidx)
    pltpu.sync_copy(dst_hbm.at[pl.ds(ebase, cpt * CH)], didx)
    plsc.subcore_barrier()

    @pl.loop(0, cpt)
    def _(j):
      pltpu.sync_copy(ones_v, hist_o.at[sidx.at[pl.ds(j * CH, CH)]],
                      add=True)
      pltpu.sync_copy(ones_v, hist_i.at[didx.at[pl.ds(j * CH, CH)]],
                      add=True)

    plsc.subcore_barrier()

    pltpu.sync_copy(hist_o.at[pl.ds(s * zslice, zslice)], zbuf)
    pltpu.sync_copy(zbuf, out_hbm.at[pl.ds((c * 2) * n_pad + s * zslice,
                                           zslice)])
    pltpu.sync_copy(hist_i.at[pl.ds(s * zslice, zslice)], zbuf)
    pltpu.sync_copy(zbuf, out_hbm.at[pl.ds((c * 2 + 1) * n_pad + s * zslice,
                                           zslice)])

  return k


def _agg_kernel(n_pad, d, cpt0, cpt1):
  zrows = n_pad // NS
  mesh = plsc.VectorSubcoreMesh(core_axis_name="c", subcore_axis_name="s")

  @functools.partial(
      pl.kernel,
      out_type=jax.ShapeDtypeStruct((NC * n_pad, d), jnp.float32),
      mesh=mesh,
      scratch_types=[
          [pltpu.VMEM((CH,), jnp.int32) for _ in range(4)],
          [pltpu.VMEM((CH,), jnp.int32) for _ in range(4)],
          [pltpu.VMEM((CH, d), jnp.float32) for _ in range(2)],
          pltpu.VMEM_SHARED((n_pad, d), jnp.float32),
          pltpu.SemaphoreType.DMA,
          [pltpu.SemaphoreType.DMA for _ in range(2)],
      ],
  )
  def k(h_hbm, src_hbm, dst_hbm, z_hbm, out_hbm, sidx, didx, rows, acc,
        lsem, gsem):
    c = lax.axis_index("c")
    s = lax.axis_index("s")
    cptc = jnp.where(c == 0, cpt0, cpt1)
    cbase = jnp.where(c == 0, s * cpt0, NS * cpt0 + s * cpt1)

    pltpu.sync_copy(z_hbm, rows[0])
    for r in range(zrows // CH):
      pltpu.sync_copy(rows[0], acc.at[pl.ds(s * zrows + r * CH, CH)])

    def load_idx(j, a, b):
      lds = []
      for i, q in enumerate(range(a, b + 1)):
        off = (cbase + j + i) * CH
        lds.append(pltpu.async_copy(src_hbm.at[pl.ds(off, CH)], sidx[q],
                                    lsem))
        lds.append(pltpu.async_copy(dst_hbm.at[pl.ds(off, CH)], didx[q],
                                    lsem))
      return lds

    for d_ in load_idx(0, 0, 1):
      d_.wait()
    plsc.subcore_barrier()

    @pl.loop(0, cptc // 4)
    def _(g):
      j = g * 4
      for ph in range(2):
        q0, q1 = (0, 1) if ph == 0 else (2, 3)
        n0, n1 = (2, 3) if ph == 0 else (0, 1)
        g0 = pltpu.async_copy(h_hbm.at[sidx[q0]], rows[0], gsem[0])
        g1 = pltpu.async_copy(h_hbm.at[sidx[q1]], rows[1], gsem[1])
        lds = load_idx(j + 2 * ph + 2, n0, n1)
        for d_ in lds:
          d_.wait()
        g0.wait()
        g1.wait()
        pltpu.sync_copy(rows[0], acc.at[didx[q0]], add=True)
        pltpu.sync_copy(rows[1], acc.at[didx[q1]], add=True)

    plsc.subcore_barrier()

    for r in range(zrows // CH):
      roff = s * zrows + r * CH
      pltpu.sync_copy(acc.at[pl.ds(roff, CH)], rows[0])
      pltpu.sync_copy(rows[0], out_hbm.at[pl.ds(c * n_pad + roff, CH)])

  return k


def _norms_body(degs_ref, out_ref):
  dego = degs_ref[0, 0] + degs_ref[1, 0]
  degi = degs_ref[0, 1] + degs_ref[1, 1]
  out_ref[0] = lax.rsqrt(jnp.maximum(dego, 1.0))
  out_ref[1] = lax.rsqrt(jnp.maximum(degi, 1.0))


def _scale_body(y_ref, ns_ref, h_ref):
  h_ref[...] = y_ref[...] * ns_ref[...]


def _final_body(acc_ref, w_ref, nd_ref, b_ref, out_ref, *, n, n_pad):
  agg = acc_ref[0:n] + acc_ref[n_pad:n_pad + n]
  y = jnp.dot(agg, w_ref[...], preferred_element_type=jnp.float32)
  y = y * nd_ref[...] + b_ref[...]
  m = jnp.max(y, axis=1, keepdims=True)
  e = jnp.exp(y - m)
  out_ref[...] = e / jnp.sum(e, axis=1, keepdims=True)


def kernel(x, edge_index, W, b):
  n, d = x.shape
  e = edge_index.shape[1]
  n_pad = ((n + 1 + NS * CH - 1) // (NS * CH)) * (NS * CH)
  cpt = 2 * -(-e // (NW * CH * 2))
  e_pad = NW * CH * cpt

  cpt0, cpt1 = cpt, cpt
  assert cpt0 % 4 == 0 and cpt1 % 4 == 0 and cpt0 + cpt1 == 2 * cpt

  src = edge_index[0].astype(jnp.int32)
  dst = edge_index[1].astype(jnp.int32)
  pad = e_pad - e + 2 * CH
  if pad:
    padv = n + jnp.arange(pad, dtype=jnp.int32) % (n_pad - n)
    src = jnp.concatenate([src, padv])
    dst = jnp.concatenate([dst, padv])

  ones_e = jnp.ones((CH,), jnp.float32)
  z1 = jnp.zeros((n_pad // NS,), jnp.float32)
  z2 = jnp.zeros((CH, d), jnp.float32)

  degs = _deg_kernel(n_pad, cpt)(src, dst, ones_e, z1)
  x_pad = jnp.concatenate([x, jnp.zeros((n_pad - n, d), jnp.float32)])

  degs4 = degs.reshape(NC, 2, n_pad // d, d)
  norms = pl.pallas_call(
      _norms_body,
      out_shape=jax.ShapeDtypeStruct((2, n_pad // d, d), jnp.float32))(degs4)
  ns_col = norms[0].reshape(-1)[:, None]
  nd_col = norms[1].reshape(-1)[:n][:, None]

  h = pl.pallas_call(
      _scale_body,
      out_shape=jax.ShapeDtypeStruct((n_pad, d), jnp.float32))(x_pad, ns_col)

  acc = _agg_kernel(n_pad, d, cpt0, cpt1)(h, src, dst, z2)

  out = pl.pallas_call(
      functools.partial(_final_body, n=n, n_pad=n_pad),
      out_shape=jax.ShapeDtypeStruct((n, d), jnp.float32))(
          acc, W, nd_col, b.reshape(1, d))
  return out


# speedup vs baseline: 1.1990x; 1.0227x over previous
import functools

import jax
import jax.numpy as jnp
from jax import lax
from jax.experimental import pallas as pl
from jax.experimental.pallas import tpu as pltpu
from jax.experimental.pallas import tpu_sc as plsc

NC = 2
NS = 16
NW = NC * NS
CH = 128


def _deg_kernel(n_pad, cpt):
  zslice = n_pad // NS
  mesh = plsc.VectorSubcoreMesh(core_axis_name="c", subcore_axis_name="s")

  @functools.partial(
      pl.kernel,
      out_type=jax.ShapeDtypeStruct((NC * 2 * n_pad,), jnp.float32),
      mesh=mesh,
      scratch_types=[
          pltpu.VMEM((zslice,), jnp.float32),
          pltpu.VMEM((CH,), jnp.float32),
          pltpu.VMEM((cpt * CH,), jnp.int32),
          pltpu.VMEM((cpt * CH,), jnp.int32),
          pltpu.VMEM_SHARED((n_pad,), jnp.float32),
          pltpu.VMEM_SHARED((n_pad,), jnp.float32),
          pltpu.SemaphoreType.DMA,
          pltpu.SemaphoreType.DMA,
      ],
  )
  def k(src_hbm, dst_hbm, ones_hbm, z_hbm, out_hbm, zbuf, ones_v, sidx,
        didx, hist_o, hist_i, osem, isem):
    c = lax.axis_index("c")
    s = lax.axis_index("s")
    wid = s * NC + c
    ebase = wid * cpt * CH

    pltpu.sync_copy(z_hbm, zbuf)
    pltpu.sync_copy(zbuf, hist_o.at[pl.ds(s * zslice, zslice)])
    pltpu.sync_copy(zbuf, hist_i.at[pl.ds(s * zslice, zslice)])
    pltpu.sync_copy(ones_hbm, ones_v)
    pltpu.sync_copy(src_hbm.at[pl.ds(ebase, cpt * CH)], sidx)
    pltpu.sync_copy(dst_hbm.at[pl.ds(ebase, cpt * CH)], didx)
    plsc.subcore_barrier()

    @pl.loop(0, cpt)
    def _(j):
      so = pltpu.async_copy(ones_v, hist_o.at[sidx.at[pl.ds(j * CH, CH)]],
                            osem, add=True)
      si = pltpu.async_copy(ones_v, hist_i.at[didx.at[pl.ds(j * CH, CH)]],
                            isem, add=True)
      so.wait()
      si.wait()

    plsc.subcore_barrier()

    pltpu.sync_copy(hist_o.at[pl.ds(s * zslice, zslice)], zbuf)
    pltpu.sync_copy(zbuf, out_hbm.at[pl.ds((c * 2) * n_pad + s * zslice,
                                           zslice)])
    pltpu.sync_copy(hist_i.at[pl.ds(s * zslice, zslice)], zbuf)
    pltpu.sync_copy(zbuf, out_hbm.at[pl.ds((c * 2 + 1) * n_pad + s * zslice,
                                           zslice)])

  return k


def _agg_kernel(n_pad, d, cpt0, cpt1):
  zrows = n_pad // NS
  mesh = plsc.VectorSubcoreMesh(core_axis_name="c", subcore_axis_name="s")

  @functools.partial(
      pl.kernel,
      out_type=jax.ShapeDtypeStruct((NC * n_pad, d), jnp.float32),
      mesh=mesh,
      scratch_types=[
          [pltpu.VMEM((CH,), jnp.int32) for _ in range(4)],
          [pltpu.VMEM((CH,), jnp.int32) for _ in range(4)],
          [pltpu.VMEM((CH, d), jnp.float32) for _ in range(2)],
          pltpu.VMEM_SHARED((n_pad, d), jnp.float32),
          pltpu.SemaphoreType.DMA,
          [pltpu.SemaphoreType.DMA for _ in range(2)],
      ],
  )
  def k(h_hbm, src_hbm, dst_hbm, z_hbm, out_hbm, sidx, didx, rows, acc,
        lsem, gsem):
    c = lax.axis_index("c")
    s = lax.axis_index("s")
    cptc = jnp.where(c == 0, cpt0, cpt1)
    cbase = jnp.where(c == 0, s * cpt0, NS * cpt0 + s * cpt1)

    pltpu.sync_copy(z_hbm, rows[0])
    for r in range(zrows // CH):
      pltpu.sync_copy(rows[0], acc.at[pl.ds(s * zrows + r * CH, CH)])

    def load_idx(j, a, b):
      lds = []
      for i, q in enumerate(range(a, b + 1)):
        off = (cbase + j + i) * CH
        lds.append(pltpu.async_copy(src_hbm.at[pl.ds(off, CH)], sidx[q],
                                    lsem))
        lds.append(pltpu.async_copy(dst_hbm.at[pl.ds(off, CH)], didx[q],
                                    lsem))
      return lds

    for d_ in load_idx(0, 0, 1):
      d_.wait()
    plsc.subcore_barrier()

    @pl.loop(0, cptc // 4)
    def _(g):
      j = g * 4
      for ph in range(2):
        q0, q1 = (0, 1) if ph == 0 else (2, 3)
        n0, n1 = (2, 3) if ph == 0 else (0, 1)
        g0 = pltpu.async_copy(h_hbm.at[sidx[q0]], rows[0], gsem[0])
        g1 = pltpu.async_copy(h_hbm.at[sidx[q1]], rows[1], gsem[1])
        lds = load_idx(j + 2 * ph + 2, n0, n1)
        for d_ in lds:
          d_.wait()
        g0.wait()
        g1.wait()
        pltpu.sync_copy(rows[0], acc.at[didx[q0]], add=True)
        pltpu.sync_copy(rows[1], acc.at[didx[q1]], add=True)

    plsc.subcore_barrier()

    for r in range(zrows // CH):
      roff = s * zrows + r * CH
      pltpu.sync_copy(acc.at[pl.ds(roff, CH)], rows[0])
      pltpu.sync_copy(rows[0], out_hbm.at[pl.ds(c * n_pad + roff, CH)])

  return k


def _norms_body(degs_ref, out_ref):
  dego = degs_ref[0, 0] + degs_ref[1, 0]
  degi = degs_ref[0, 1] + degs_ref[1, 1]
  out_ref[0] = lax.rsqrt(jnp.maximum(dego, 1.0))
  out_ref[1] = lax.rsqrt(jnp.maximum(degi, 1.0))


def _scale_body(y_ref, ns_ref, h_ref):
  h_ref[...] = y_ref[...] * ns_ref[...]


def _final_body(acc_ref, w_ref, nd_ref, b_ref, out_ref, *, n, n_pad):
  agg = acc_ref[0:n] + acc_ref[n_pad:n_pad + n]
  y = jnp.dot(agg, w_ref[...], preferred_element_type=jnp.float32)
  y = y * nd_ref[...] + b_ref[...]
  m = jnp.max(y, axis=1, keepdims=True)
  e = jnp.exp(y - m)
  out_ref[...] = e / jnp.sum(e, axis=1, keepdims=True)


def kernel(x, edge_index, W, b):
  n, d = x.shape
  e = edge_index.shape[1]
  n_pad = ((n + 1 + NS * CH - 1) // (NS * CH)) * (NS * CH)
  cpt = 2 * -(-e // (NW * CH * 2))
  e_pad = NW * CH * cpt

  cpt0, cpt1 = cpt, cpt
  assert cpt0 % 4 == 0 and cpt1 % 4 == 0 and cpt0 + cpt1 == 2 * cpt

  src = edge_index[0].astype(jnp.int32)
  dst = edge_index[1].astype(jnp.int32)
  pad = e_pad - e + 2 * CH
  if pad:
    padv = n + jnp.arange(pad, dtype=jnp.int32) % (n_pad - n)
    src = jnp.concatenate([src, padv])
    dst = jnp.concatenate([dst, padv])

  ones_e = jnp.ones((CH,), jnp.float32)
  z1 = jnp.zeros((n_pad // NS,), jnp.float32)
  z2 = jnp.zeros((CH, d), jnp.float32)

  degs = _deg_kernel(n_pad, cpt)(src, dst, ones_e, z1)
  x_pad = jnp.concatenate([x, jnp.zeros((n_pad - n, d), jnp.float32)])

  degs4 = degs.reshape(NC, 2, n_pad // d, d)
  norms = pl.pallas_call(
      _norms_body,
      out_shape=jax.ShapeDtypeStruct((2, n_pad // d, d), jnp.float32))(degs4)
  ns_col = norms[0].reshape(-1)[:, None]
  nd_col = norms[1].reshape(-1)[:n][:, None]

  h = pl.pallas_call(
      _scale_body,
      out_shape=jax.ShapeDtypeStruct((n_pad, d), jnp.float32))(x_pad, ns_col)

  acc = _agg_kernel(n_pad, d, cpt0, cpt1)(h, src, dst, z2)

  out = pl.pallas_call(
      functools.partial(_final_body, n=n, n_pad=n_pad),
      out_shape=jax.ShapeDtypeStruct((n, d), jnp.float32))(
          acc, W, nd_col, b.reshape(1, d))
  return out
